# baseline (speedup 1.0000x reference)
import jax
import jax.numpy as jnp
from jax import lax
from jax.experimental import pallas as pl
from jax.experimental.pallas import tpu as pltpu
from jax.experimental.pallas import tpu_sc as plsc

MAX_REL = 64
VOCAB = 2 * MAX_REL + 1
ROW = 128
N = 512
LANES = 16
NR = 8
SW = 1024


def _strips_body(table_hbm, strips_hbm, idx_v, buf_v, gsem):
    nc = plsc.get_sparse_core_info().num_cores
    ns = plsc.get_sparse_core_info().num_subcores
    wid = lax.axis_index("s") * nc + lax.axis_index("c")
    r = wid // 2
    h = wid % 2

    @pl.when(wid < 2 * NR)
    def _build():
        lane = lax.iota(jnp.int32, LANES)
        for c in range(4):
            for j in range(128 // LANES):
                m = lane + j * LANES + c * 128 + h * 512
                idx_v[pl.ds(j * LANES, LANES)] = (
                    jnp.clip(m - (N - NR) - r, -MAX_REL, MAX_REL) + MAX_REL)
            pltpu.async_copy(table_hbm.at[idx_v],
                             buf_v.at[pl.ds(c * 128, 128)], gsem).wait()
        pltpu.sync_copy(buf_v, strips_hbm.at[r, pl.ds(h * 512, 512)])


def _expand_body(strips_ref, out_ref):
    t = pl.program_id(1)
    off = pl.multiple_of(8 * (63 - t), 8)
    w = strips_ref[0, pl.ds(off, N), :]
    out_ref[0] = jnp.transpose(w, (1, 0))


def kernel(len_in, len_out, table):
    del len_in, len_out
    mesh = plsc.VectorSubcoreMesh(core_axis_name="c", subcore_axis_name="s")
    build = pl.kernel(
        _strips_body,
        mesh=mesh,
        out_type=jax.ShapeDtypeStruct((NR, SW, ROW), jnp.float32),
        scratch_types=[
            pltpu.VMEM((128,), jnp.int32),
            pltpu.VMEM((512, ROW), jnp.float32),
            pltpu.SemaphoreType.DMA,
        ],
    )
    strips = build(table)

    out_phys = pl.pallas_call(
        _expand_body,
        grid=(NR, N // NR),
        in_specs=[pl.BlockSpec((1, SW, ROW), lambda r, t: (r, 0, 0))],
        out_specs=pl.BlockSpec((1, ROW, N), lambda r, t: (NR * t + r, 0, 0)),
        out_shape=jax.ShapeDtypeStruct((N, ROW, N), jnp.float32),
    )(strips)

    return jnp.transpose(out_phys.reshape(N, 8, 16, N), (0, 3, 1, 2))
